# baseline (speedup 1.0000x reference)
import jax
import jax.numpy as jnp
from jax.experimental import pallas as pl

_BN = 1024


def _tmk_block(x_ref, pts_t_ref, c_ref, out_ref):
    D = x_ref.shape[1]
    acc = None
    for d in range(D):
        t = jnp.abs(x_ref[:, d : d + 1] - pts_t_ref[d : d + 1, :])
        acc = t if acc is None else acc + t
    k = jnp.exp(-acc)
    out_ref[...] = jnp.dot(k, c_ref[...], preferred_element_type=jnp.float32)


def kernel(input, pts_set, chol_inv):
    N, D = input.shape
    M = pts_set.shape[0]
    pts_t = pts_set.T
    return pl.pallas_call(
        _tmk_block,
        grid=(N // _BN,),
        in_specs=[
            pl.BlockSpec((_BN, D), lambda i: (i, 0)),
            pl.BlockSpec((D, M), lambda i: (0, 0)),
            pl.BlockSpec((M, M), lambda i: (0, 0)),
        ],
        out_specs=pl.BlockSpec((_BN, M), lambda i: (i, 0)),
        out_shape=jax.ShapeDtypeStruct((N, M), jnp.float32),
    )(input, pts_t, chol_inv)
